# baseline (speedup 1.0000x reference)
import functools

import jax
import jax.numpy as jnp
from jax import lax
from jax.experimental import pallas as pl
from jax.experimental.pallas import tpu as pltpu
from jax.experimental.pallas import tpu_sc as plsc

N_NODES = 10000
N_EDGES = 2500
NNZ = 320000
D = 128

NC, NS = 2, 16
NW = NC * NS
PER_W = NNZ // NW
CH_E = 64
NCH_E = -(-PER_W // CH_E)
CH_V = 64
NCH_V = -(-PER_W // CH_V)
E_PAD = 2560
V_PAD = 10112
DEG_W = 16

_MESH = plsc.VectorSubcoreMesh(core_axis_name="c", subcore_axis_name="s")


def _zero_stripe(buf, ch, d, acc, base, stripe):
    zeros = jnp.zeros((16,), jnp.float32)

    def zrow(i, carry):
        for j in range(d // 16):
            buf[i, pl.ds(j * 16, 16)] = zeros
        return carry

    lax.fori_loop(0, ch, zrow, 0)
    off = 0
    while off < stripe:
        n = min(ch, stripe - off)
        pltpu.sync_copy(buf.at[pl.ds(0, n)], acc.at[pl.ds(base + off, n)])
        off += n


def _seg_loop(tbl, gv, sv, buf_a, buf_b, acc, sem_a, sem_b, nch):
    pltpu.async_copy(tbl.at[gv.at[0]], buf_a, sem_a)

    def chunk(i, carry):
        j0 = i * 2
        pltpu.async_copy(tbl.at[gv.at[j0 + 1]], buf_b, sem_b)
        pltpu.make_async_copy(tbl.at[gv.at[j0]], buf_a, sem_a).wait()
        pltpu.sync_copy(buf_a, acc.at[sv.at[j0]], add=True)

        @pl.when(j0 + 2 < nch)
        def _():
            pltpu.async_copy(tbl.at[gv.at[j0 + 2]], buf_a, sem_a)

        pltpu.make_async_copy(tbl.at[gv.at[j0 + 1]], buf_b, sem_b).wait()
        pltpu.sync_copy(buf_b, acc.at[sv.at[j0 + 1]], add=True)
        return carry

    lax.fori_loop(0, nch // 2, chunk, 0)
    if nch % 2:
        j = nch - 1
        pltpu.make_async_copy(tbl.at[gv.at[j]], buf_a, sem_a).wait()
        pltpu.sync_copy(buf_a, acc.at[sv.at[j]], add=True)


def _make_seg_sum(n_pad, nch, ch):
    stripe = n_pad // NS

    @functools.partial(
        pl.kernel,
        out_type=jax.ShapeDtypeStruct((NC, n_pad, D), jnp.float32),
        mesh=_MESH,
        scratch_types=[
            pltpu.VMEM((nch, ch), jnp.int32),
            pltpu.VMEM((nch, ch), jnp.int32),
            pltpu.VMEM((ch, D), jnp.float32),
            pltpu.VMEM((ch, D), jnp.float32),
            pltpu.VMEM_SHARED((n_pad, D), jnp.float32),
            pltpu.SemaphoreType.DMA,
            pltpu.SemaphoreType.DMA,
        ],
        compiler_params=pltpu.CompilerParams(use_tc_tiling_on_sc=False),
    )
    def seg_sum(table_hbm, gidx_hbm, sidx_hbm, out_hbm, gv, sv, buf_a, buf_b,
                acc, sem_a, sem_b):
        c = lax.axis_index("c")
        s = lax.axis_index("s")
        pltpu.sync_copy(gidx_hbm.at[c, s], gv)
        pltpu.sync_copy(sidx_hbm.at[c, s], sv)
        base = s * stripe
        _zero_stripe(buf_a, ch, D, acc, base, stripe)
        plsc.subcore_barrier()
        _seg_loop(table_hbm, gv, sv, buf_a, buf_b, acc, sem_a, sem_b, nch)
        plsc.subcore_barrier()
        pltpu.sync_copy(acc.at[pl.ds(base, stripe)],
                        out_hbm.at[c, pl.ds(base, stripe)])

    return seg_sum


_SEG_E = _make_seg_sum(E_PAD, NCH_E, CH_E)
_SEG_V = _make_seg_sum(V_PAD, NCH_V, CH_V)


@functools.partial(
    pl.kernel,
    out_type=(jax.ShapeDtypeStruct((NC, E_PAD, DEG_W), jnp.float32),
              jax.ShapeDtypeStruct((NC, V_PAD, DEG_W), jnp.float32)),
    mesh=_MESH,
    scratch_types=[
        pltpu.VMEM((NCH_E, CH_E), jnp.int32),
        pltpu.VMEM((NCH_E, CH_E), jnp.int32),
        pltpu.VMEM((CH_E, DEG_W), jnp.float32),
        pltpu.VMEM((CH_E, DEG_W), jnp.float32),
        pltpu.VMEM_SHARED((E_PAD, DEG_W), jnp.float32),
        pltpu.VMEM_SHARED((V_PAD, DEG_W), jnp.float32),
    ],
    compiler_params=pltpu.CompilerParams(use_tc_tiling_on_sc=False),
)
def _degrees(eidx_hbm, nidx_hbm, oute_hbm, outv_hbm, ev, nv, ones_b, zero_b,
             acc_e, acc_v):
    c = lax.axis_index("c")
    s = lax.axis_index("s")
    pltpu.sync_copy(eidx_hbm.at[c, s], ev)
    pltpu.sync_copy(nidx_hbm.at[c, s], nv)

    ones = jnp.ones((16,), jnp.float32)
    zeros = jnp.zeros((16,), jnp.float32)

    def fill(i, carry):
        ones_b[i, pl.ds(0, DEG_W)] = ones
        zero_b[i, pl.ds(0, DEG_W)] = zeros
        return carry

    lax.fori_loop(0, CH_E, fill, 0)
    se = E_PAD // NS
    sv_ = V_PAD // NS
    for base, stripe, acc in ((s * se, se, acc_e), (s * sv_, sv_, acc_v)):
        off = 0
        while off < stripe:
            n = min(CH_E, stripe - off)
            pltpu.sync_copy(zero_b.at[pl.ds(0, n)], acc.at[pl.ds(base + off, n)])
            off += n
    plsc.subcore_barrier()

    def chunk(j, carry):
        pltpu.sync_copy(ones_b, acc_e.at[ev.at[j]], add=True)
        pltpu.sync_copy(ones_b, acc_v.at[nv.at[j]], add=True)
        return carry

    lax.fori_loop(0, NCH_E, chunk, 0)
    plsc.subcore_barrier()
    pltpu.sync_copy(acc_e.at[pl.ds(s * se, se)],
                    oute_hbm.at[c, pl.ds(s * se, se)])
    pltpu.sync_copy(acc_v.at[pl.ds(s * sv_, sv_)],
                    outv_hbm.at[c, pl.ds(s * sv_, sv_)])


_NODE_BLK = 1000
_NODE_GRID = N_NODES // _NODE_BLK


def _tc_ft_body(x, wft, bft, lng, lnb, w1, b1, out):
    h = jnp.dot(x[...], wft[...], preferred_element_type=jnp.float32) + bft[...]
    h = jax.nn.gelu(h)
    m = jnp.mean(h, axis=-1, keepdims=True)
    var = jnp.mean((h - m) * (h - m), axis=-1, keepdims=True)
    h = (h - m) / jnp.sqrt(var + 1e-5) * lng[...] + lnb[...]
    out[...] = jnp.dot(h, w1[...], preferred_element_type=jnp.float32) + b1[...]


def _tc_ft(X, wft, bft, lng, lnb, w1, b1):
    full = lambda i: (0, 0)
    return pl.pallas_call(
        _tc_ft_body,
        grid=(_NODE_GRID,),
        in_specs=[
            pl.BlockSpec((_NODE_BLK, D), lambda i: (i, 0)),
            pl.BlockSpec((D, D), full),
            pl.BlockSpec((1, D), full),
            pl.BlockSpec((1, D), full),
            pl.BlockSpec((1, D), full),
            pl.BlockSpec((D, D), full),
            pl.BlockSpec((1, D), full),
        ],
        out_specs=pl.BlockSpec((_NODE_BLK, D), lambda i: (i, 0)),
        out_shape=jax.ShapeDtypeStruct((N_NODES, D), jnp.float32),
    )(X, wft, bft, lng, lnb, w1, b1)


def _tc_edge_body(has_prev, *refs):
    if has_prev:
        ep, dp, eprev, watt, batt, out = refs
    else:
        ep, dp, watt, batt, out = refs
    deg = jnp.clip(dp[0, :, 0:1] + dp[1, :, 0:1], 1.0, None)
    ef = (ep[0] + ep[1]) / deg
    if has_prev:
        ef = ef + eprev[...]
    a = jax.nn.sigmoid(
        jnp.dot(ef, watt[...], preferred_element_type=jnp.float32) + batt[...])
    out[...] = ef * a


def _tc_edge(ep, dp, eprev, watt, batt):
    args = [ep, dp] + ([eprev] if eprev is not None else []) + [watt, batt]
    return pl.pallas_call(
        functools.partial(_tc_edge_body, eprev is not None),
        out_shape=jax.ShapeDtypeStruct((E_PAD, D), jnp.float32),
    )(*args)


def _tc_node_body(vp, dvp, xt, w, b, out):
    deg = jnp.clip(dvp[0, :, 0:1] + dvp[1, :, 0:1], 1.0, None)
    v = (vp[0] + vp[1]) / deg + xt[...]
    v = jax.nn.gelu(v)
    out[...] = jnp.dot(v, w[...], preferred_element_type=jnp.float32) + b[...]


def _tc_node(vp, dvp, xt, w, b):
    full = lambda i: (0, 0)
    return pl.pallas_call(
        _tc_node_body,
        grid=(_NODE_GRID,),
        in_specs=[
            pl.BlockSpec((2, _NODE_BLK, D), lambda i: (0, i, 0)),
            pl.BlockSpec((2, _NODE_BLK, DEG_W), lambda i: (0, i, 0)),
            pl.BlockSpec((_NODE_BLK, D), lambda i: (i, 0)),
            pl.BlockSpec((D, D), full),
            pl.BlockSpec((1, D), full),
        ],
        out_specs=pl.BlockSpec((_NODE_BLK, D), lambda i: (i, 0)),
        out_shape=jax.ShapeDtypeStruct((N_NODES, D), jnp.float32),
    )(vp, dvp, xt, w, b)


def _tc_final_body(ep, dp, e2, watt, batt, wr, br, w1c, b1c, w2b, b2v,
                   wf1, bf1, wf2, bf2, bnm, bnv, bng, bnb, wo, bo,
                   score_out, att_out):
    deg = jnp.clip(dp[0, :, 0:1] + dp[1, :, 0:1], 1.0, None)
    ef = (ep[0] + ep[1]) / deg + e2[...]
    a3 = jax.nn.sigmoid(
        jnp.dot(ef, watt[...], preferred_element_type=jnp.float32) + batt[...])
    e3 = ef * a3
    refined = jax.nn.gelu(
        jnp.dot(e3, wr[...], preferred_element_type=jnp.float32) + br[...])
    t = jax.nn.gelu(
        jnp.dot(refined, w1c[...], preferred_element_type=jnp.float32) + b1c[...])
    combined = jnp.dot(t, w2b[...], preferred_element_type=jnp.float32) + b2v[...]
    aw = jax.nn.sigmoid(jnp.mean(combined, axis=1, keepdims=True))
    fatt = (aw + a3) * 0.5
    xw = refined * fatt
    t1 = jax.nn.gelu(
        jnp.dot(xw, wf1[...], preferred_element_type=jnp.float32) + bf1[...])
    xe = jax.nn.gelu(
        jnp.dot(t1, wf2[...], preferred_element_type=jnp.float32) + bf2[...])
    xs = xe + xw
    xs = (xs - bnm[...]) / jnp.sqrt(bnv[...] + 1e-5) * bng[...] + bnb[...]
    score_out[...] = jax.nn.sigmoid(
        jnp.dot(xs, wo[...], preferred_element_type=jnp.float32) + bo[...])
    att_out[...] = fatt


def _tc_final(ep, dp, e2, watt, batt, p):
    w1c = jnp.concatenate([hp["l1"]["W"] for hp in p["heads"]], axis=1)
    b1c = jnp.concatenate([hp["l1"]["b"] for hp in p["heads"]])[None, :]
    w2b = jax.scipy.linalg.block_diag(*[hp["l2"]["W"] for hp in p["heads"]])
    b2v = jnp.stack([hp["l2"]["b"][0] for hp in p["heads"]])[None, :]
    row = lambda a: a[None, :]
    return pl.pallas_call(
        _tc_final_body,
        out_shape=(jax.ShapeDtypeStruct((E_PAD, 1), jnp.float32),
                   jax.ShapeDtypeStruct((E_PAD, 1), jnp.float32)),
    )(ep, dp, e2, watt, batt,
      p["refine"]["W"], row(p["refine"]["b"]), w1c, b1c, w2b, b2v,
      p["fe1"]["W"], row(p["fe1"]["b"]), p["fe2"]["W"], row(p["fe2"]["b"]),
      row(p["bn_m"]), row(p["bn_v"]), row(p["bn_g"]), row(p["bn_b"]),
      p["out"]["W"], row(p["out"]["b"]))


def kernel(X, node_idx, edge_idx, params):
    p = params

    def _padded(idx, nch, ch, fill):
        a = idx.astype(jnp.int32).reshape(NW, PER_W)
        pad = jnp.full((NW, nch * ch - PER_W), fill, jnp.int32)
        return jnp.concatenate([a, pad], axis=1).reshape(NC, NS, nch, ch)

    nidx_g = _padded(node_idx, NCH_E, CH_E, 0)
    nidx_sE = _padded(node_idx, NCH_E, CH_E, V_PAD - 1)
    nidx_s = _padded(node_idx, NCH_V, CH_V, V_PAD - 1)
    eidx_g = _padded(edge_idx, NCH_V, CH_V, 0)
    eidx_s = _padded(edge_idx, NCH_E, CH_E, E_PAD - 1)
    row = lambda a: a[None, :]

    dp_e, dp_v = _degrees(eidx_s, nidx_sE)

    xt1 = _tc_ft(X, p["ft"]["W"], row(p["ft"]["b"]), row(p["ln_g"]),
                 row(p["ln_b"]), p["conv1"]["W"], row(p["conv1"]["b"]))

    ep1 = _SEG_E(xt1, nidx_g, eidx_s)
    e1 = _tc_edge(ep1, dp_e, None, p["conv1"]["w_att"], row(p["conv1"]["b_att"]))
    vp1 = _SEG_V(e1, eidx_g, nidx_s)
    xt2 = _tc_node(vp1, dp_v, xt1, p["conv2"]["W"], row(p["conv2"]["b"]))

    ep2 = _SEG_E(xt2, nidx_g, eidx_s)
    e2 = _tc_edge(ep2, dp_e, e1, p["conv2"]["w_att"], row(p["conv2"]["b_att"]))
    vp2 = _SEG_V(e2, eidx_g, nidx_s)
    xt3 = _tc_node(vp2, dp_v, xt2, p["conv3"]["W"], row(p["conv3"]["b"]))

    ep3 = _SEG_E(xt3, nidx_g, eidx_s)
    score, fatt = _tc_final(ep3, dp_e, e2, p["conv3"]["w_att"],
                            row(p["conv3"]["b_att"]), p)
    return score[:N_EDGES], fatt[:N_EDGES]


# speedup vs baseline: 1.4937x; 1.4937x over previous
import functools

import jax
import jax.numpy as jnp
from jax import lax
from jax.experimental import pallas as pl
from jax.experimental.pallas import tpu as pltpu
from jax.experimental.pallas import tpu_sc as plsc

N_NODES = 10000
N_EDGES = 2500
NNZ = 320000
D = 128

NC, NS = 2, 16
NW = NC * NS
PER_W = NNZ // NW
CH_E = 80
NCH_E = -(-PER_W // CH_E)
CH_V = 80
NCH_V = -(-PER_W // CH_V)
E_PAD = 2560
V_PAD = 10112
DEG_W = 16

_MESH = plsc.VectorSubcoreMesh(core_axis_name="c", subcore_axis_name="s")


def _zero_stripe(buf, ch, d, acc, base, stripe):
    zeros = jnp.zeros((16,), jnp.float32)

    def zrow(i, carry):
        for j in range(d // 16):
            buf[i, pl.ds(j * 16, 16)] = zeros
        return carry

    lax.fori_loop(0, ch, zrow, 0)
    off = 0
    while off < stripe:
        n = min(ch, stripe - off)
        pltpu.sync_copy(buf.at[pl.ds(0, n)], acc.at[pl.ds(base + off, n)])
        off += n


def _seg_loop(tbl, gv, sv, buf_a, buf_b, acc, sem_a, sem_b, nch):
    pltpu.async_copy(tbl.at[gv.at[0]], buf_a, sem_a)

    def chunk(i, carry):
        j0 = i * 2
        pltpu.async_copy(tbl.at[gv.at[j0 + 1]], buf_b, sem_b)
        pltpu.make_async_copy(tbl.at[gv.at[j0]], buf_a, sem_a).wait()
        pltpu.sync_copy(buf_a, acc.at[sv.at[j0]], add=True)

        @pl.when(j0 + 2 < nch)
        def _():
            pltpu.async_copy(tbl.at[gv.at[j0 + 2]], buf_a, sem_a)

        pltpu.make_async_copy(tbl.at[gv.at[j0 + 1]], buf_b, sem_b).wait()
        pltpu.sync_copy(buf_b, acc.at[sv.at[j0 + 1]], add=True)
        return carry

    lax.fori_loop(0, nch // 2, chunk, 0)
    if nch % 2:
        j = nch - 1
        pltpu.make_async_copy(tbl.at[gv.at[j]], buf_a, sem_a).wait()
        pltpu.sync_copy(buf_a, acc.at[sv.at[j]], add=True)


def _seg_loop_ring(tbl, gv, sv, bufs, sems_g, sems_s, acc, nch):
    nb = len(bufs)
    pltpu.async_copy(tbl.at[gv.at[0]], bufs[0], sems_g[0])
    nsteps = -(-nch // nb)

    def step(i, carry):
        for p in range(nb):
            t = i * nb + p
            pn = (p + 1) % nb

            @pl.when(t < nch)
            def _():
                @pl.when(jnp.logical_and(t + 1 < nch, t - (nb - 1) >= 0))
                def _():
                    pltpu.make_async_copy(
                        bufs[pn], acc.at[sv.at[t - (nb - 1)]],
                        sems_s[pn]).wait()

                @pl.when(t + 1 < nch)
                def _():
                    pltpu.async_copy(tbl.at[gv.at[t + 1]], bufs[pn],
                                     sems_g[pn])

                pltpu.make_async_copy(tbl.at[gv.at[t]], bufs[p],
                                      sems_g[p]).wait()
                pltpu.async_copy(bufs[p], acc.at[sv.at[t]], sems_s[p],
                                 add=True)
        return carry

    lax.fori_loop(0, nsteps, step, 0)
    for k in range(max(0, nch - nb), nch):
        pltpu.make_async_copy(bufs[k % nb], acc.at[sv.at[k]],
                              sems_s[k % nb]).wait()


def _make_seg_sum(n_pad, nch, ch, nb):
    stripe = n_pad // NS

    @functools.partial(
        pl.kernel,
        out_type=jax.ShapeDtypeStruct((NC, n_pad, D), jnp.float32),
        mesh=_MESH,
        scratch_types=(
            [pltpu.VMEM((nch, ch), jnp.int32),
             pltpu.VMEM((nch, ch), jnp.int32)]
            + [pltpu.VMEM((ch, D), jnp.float32)] * nb
            + [pltpu.VMEM_SHARED((n_pad, D), jnp.float32)]
            + [pltpu.SemaphoreType.DMA] * (2 * nb)
        ),
        compiler_params=pltpu.CompilerParams(use_tc_tiling_on_sc=False),
    )
    def seg_sum(table_hbm, gidx_hbm, sidx_hbm, out_hbm, gv, sv, *rest):
        bufs = rest[:nb]
        acc = rest[nb]
        sems_g = rest[nb + 1:nb + 1 + nb]
        sems_s = rest[nb + 1 + nb:]
        c = lax.axis_index("c")
        s = lax.axis_index("s")
        pltpu.sync_copy(gidx_hbm.at[c, s], gv)
        pltpu.sync_copy(sidx_hbm.at[c, s], sv)
        base = s * stripe
        _zero_stripe(bufs[0], ch, D, acc, base, stripe)
        plsc.subcore_barrier()
        if nb == 2:
            _seg_loop(table_hbm, gv, sv, bufs[0], bufs[1], acc,
                      sems_g[0], sems_g[1], nch)
        else:
            _seg_loop_ring(table_hbm, gv, sv, bufs, sems_g, sems_s, acc, nch)
        plsc.subcore_barrier()
        pltpu.sync_copy(acc.at[pl.ds(base, stripe)],
                        out_hbm.at[c, pl.ds(base, stripe)])

    return seg_sum


_SEG_E = _make_seg_sum(E_PAD, NCH_E, CH_E, 2)
_SEG_V = _make_seg_sum(V_PAD, NCH_V, CH_V, 2)


@functools.partial(
    pl.kernel,
    out_type=(jax.ShapeDtypeStruct((NC, E_PAD, DEG_W), jnp.float32),
              jax.ShapeDtypeStruct((NC, V_PAD, DEG_W), jnp.float32)),
    mesh=_MESH,
    scratch_types=[
        pltpu.VMEM((NCH_E, CH_E), jnp.int32),
        pltpu.VMEM((NCH_E, CH_E), jnp.int32),
        pltpu.VMEM((CH_E, DEG_W), jnp.float32),
        pltpu.VMEM((CH_E, DEG_W), jnp.float32),
        pltpu.VMEM_SHARED((E_PAD, DEG_W), jnp.float32),
        pltpu.VMEM_SHARED((V_PAD, DEG_W), jnp.float32),
    ],
    compiler_params=pltpu.CompilerParams(use_tc_tiling_on_sc=False),
)
def _degrees(eidx_hbm, nidx_hbm, oute_hbm, outv_hbm, ev, nv, ones_b, zero_b,
             acc_e, acc_v):
    c = lax.axis_index("c")
    s = lax.axis_index("s")
    pltpu.sync_copy(eidx_hbm.at[c, s], ev)
    pltpu.sync_copy(nidx_hbm.at[c, s], nv)

    ones = jnp.ones((16,), jnp.float32)
    zeros = jnp.zeros((16,), jnp.float32)

    def fill(i, carry):
        ones_b[i, pl.ds(0, DEG_W)] = ones
        zero_b[i, pl.ds(0, DEG_W)] = zeros
        return carry

    lax.fori_loop(0, CH_E, fill, 0)
    se = E_PAD // NS
    sv_ = V_PAD // NS
    for base, stripe, acc in ((s * se, se, acc_e), (s * sv_, sv_, acc_v)):
        off = 0
        while off < stripe:
            n = min(CH_E, stripe - off)
            pltpu.sync_copy(zero_b.at[pl.ds(0, n)], acc.at[pl.ds(base + off, n)])
            off += n
    plsc.subcore_barrier()

    def chunk(j, carry):
        pltpu.sync_copy(ones_b, acc_e.at[ev.at[j]], add=True)
        pltpu.sync_copy(ones_b, acc_v.at[nv.at[j]], add=True)
        return carry

    lax.fori_loop(0, NCH_E, chunk, 0)
    plsc.subcore_barrier()
    pltpu.sync_copy(acc_e.at[pl.ds(s * se, se)],
                    oute_hbm.at[c, pl.ds(s * se, se)])
    pltpu.sync_copy(acc_v.at[pl.ds(s * sv_, sv_)],
                    outv_hbm.at[c, pl.ds(s * sv_, sv_)])


_NODE_BLK = 1000
_NODE_GRID = N_NODES // _NODE_BLK


def _tc_ft_body(x, wft, bft, lng, lnb, w1, b1, out):
    h = jnp.dot(x[...], wft[...], preferred_element_type=jnp.float32) + bft[...]
    h = jax.nn.gelu(h)
    m = jnp.mean(h, axis=-1, keepdims=True)
    var = jnp.mean((h - m) * (h - m), axis=-1, keepdims=True)
    h = (h - m) / jnp.sqrt(var + 1e-5) * lng[...] + lnb[...]
    out[...] = jnp.dot(h, w1[...], preferred_element_type=jnp.float32) + b1[...]


def _tc_ft(X, wft, bft, lng, lnb, w1, b1):
    full = lambda i: (0, 0)
    return pl.pallas_call(
        _tc_ft_body,
        grid=(_NODE_GRID,),
        in_specs=[
            pl.BlockSpec((_NODE_BLK, D), lambda i: (i, 0)),
            pl.BlockSpec((D, D), full),
            pl.BlockSpec((1, D), full),
            pl.BlockSpec((1, D), full),
            pl.BlockSpec((1, D), full),
            pl.BlockSpec((D, D), full),
            pl.BlockSpec((1, D), full),
        ],
        out_specs=pl.BlockSpec((_NODE_BLK, D), lambda i: (i, 0)),
        out_shape=jax.ShapeDtypeStruct((N_NODES, D), jnp.float32),
    )(X, wft, bft, lng, lnb, w1, b1)


def _tc_edge_body(has_prev, *refs):
    if has_prev:
        ep, dp, eprev, watt, batt, out = refs
    else:
        ep, dp, watt, batt, out = refs
    deg = jnp.clip(dp[0, :, 0:1] + dp[1, :, 0:1], 1.0, None)
    ef = (ep[0] + ep[1]) / deg
    if has_prev:
        ef = ef + eprev[...]
    a = jax.nn.sigmoid(
        jnp.dot(ef, watt[...], preferred_element_type=jnp.float32) + batt[...])
    out[...] = ef * a


def _tc_edge(ep, dp, eprev, watt, batt):
    args = [ep, dp] + ([eprev] if eprev is not None else []) + [watt, batt]
    return pl.pallas_call(
        functools.partial(_tc_edge_body, eprev is not None),
        out_shape=jax.ShapeDtypeStruct((E_PAD, D), jnp.float32),
    )(*args)


def _tc_node_body(vp, dvp, xt, w, b, out):
    deg = jnp.clip(dvp[0, :, 0:1] + dvp[1, :, 0:1], 1.0, None)
    v = (vp[0] + vp[1]) / deg + xt[...]
    v = jax.nn.gelu(v)
    out[...] = jnp.dot(v, w[...], preferred_element_type=jnp.float32) + b[...]


def _tc_node(vp, dvp, xt, w, b):
    full = lambda i: (0, 0)
    return pl.pallas_call(
        _tc_node_body,
        grid=(_NODE_GRID,),
        in_specs=[
            pl.BlockSpec((2, _NODE_BLK, D), lambda i: (0, i, 0)),
            pl.BlockSpec((2, _NODE_BLK, DEG_W), lambda i: (0, i, 0)),
            pl.BlockSpec((_NODE_BLK, D), lambda i: (i, 0)),
            pl.BlockSpec((D, D), full),
            pl.BlockSpec((1, D), full),
        ],
        out_specs=pl.BlockSpec((_NODE_BLK, D), lambda i: (i, 0)),
        out_shape=jax.ShapeDtypeStruct((N_NODES, D), jnp.float32),
    )(vp, dvp, xt, w, b)


def _tc_final_body(ep, dp, e2, watt, batt, wr, br, w1c, b1c, w2b, b2v,
                   wf1, bf1, wf2, bf2, bnm, bnv, bng, bnb, wo, bo,
                   score_out, att_out):
    deg = jnp.clip(dp[0, :, 0:1] + dp[1, :, 0:1], 1.0, None)
    ef = (ep[0] + ep[1]) / deg + e2[...]
    a3 = jax.nn.sigmoid(
        jnp.dot(ef, watt[...], preferred_element_type=jnp.float32) + batt[...])
    e3 = ef * a3
    refined = jax.nn.gelu(
        jnp.dot(e3, wr[...], preferred_element_type=jnp.float32) + br[...])
    t = jax.nn.gelu(
        jnp.dot(refined, w1c[...], preferred_element_type=jnp.float32) + b1c[...])
    combined = jnp.dot(t, w2b[...], preferred_element_type=jnp.float32) + b2v[...]
    aw = jax.nn.sigmoid(jnp.mean(combined, axis=1, keepdims=True))
    fatt = (aw + a3) * 0.5
    xw = refined * fatt
    t1 = jax.nn.gelu(
        jnp.dot(xw, wf1[...], preferred_element_type=jnp.float32) + bf1[...])
    xe = jax.nn.gelu(
        jnp.dot(t1, wf2[...], preferred_element_type=jnp.float32) + bf2[...])
    xs = xe + xw
    xs = (xs - bnm[...]) / jnp.sqrt(bnv[...] + 1e-5) * bng[...] + bnb[...]
    score_out[...] = jax.nn.sigmoid(
        jnp.dot(xs, wo[...], preferred_element_type=jnp.float32) + bo[...])
    att_out[...] = fatt


def _tc_final(ep, dp, e2, watt, batt, p):
    w1c = jnp.concatenate([hp["l1"]["W"] for hp in p["heads"]], axis=1)
    b1c = jnp.concatenate([hp["l1"]["b"] for hp in p["heads"]])[None, :]
    w2b = jax.scipy.linalg.block_diag(*[hp["l2"]["W"] for hp in p["heads"]])
    b2v = jnp.stack([hp["l2"]["b"][0] for hp in p["heads"]])[None, :]
    row = lambda a: a[None, :]
    return pl.pallas_call(
        _tc_final_body,
        out_shape=(jax.ShapeDtypeStruct((E_PAD, 1), jnp.float32),
                   jax.ShapeDtypeStruct((E_PAD, 1), jnp.float32)),
    )(ep, dp, e2, watt, batt,
      p["refine"]["W"], row(p["refine"]["b"]), w1c, b1c, w2b, b2v,
      p["fe1"]["W"], row(p["fe1"]["b"]), p["fe2"]["W"], row(p["fe2"]["b"]),
      row(p["bn_m"]), row(p["bn_v"]), row(p["bn_g"]), row(p["bn_b"]),
      p["out"]["W"], row(p["out"]["b"]))


def kernel(X, node_idx, edge_idx, params):
    p = params

    def _padded(idx, nch, ch, fill):
        a = idx.astype(jnp.int32).reshape(NW, PER_W)
        pad = jnp.full((NW, nch * ch - PER_W), fill, jnp.int32)
        return jnp.concatenate([a, pad], axis=1).reshape(NC, NS, nch, ch)

    nidx_g = _padded(node_idx, NCH_E, CH_E, 0)
    nidx_sE = _padded(node_idx, NCH_E, CH_E, V_PAD - 1)
    nidx_s = _padded(node_idx, NCH_V, CH_V, V_PAD - 1)
    eidx_g = _padded(edge_idx, NCH_V, CH_V, 0)
    eidx_s = _padded(edge_idx, NCH_E, CH_E, E_PAD - 1)
    row = lambda a: a[None, :]

    dp_e, dp_v = _degrees(eidx_s, nidx_sE)

    xt1 = _tc_ft(X, p["ft"]["W"], row(p["ft"]["b"]), row(p["ln_g"]),
                 row(p["ln_b"]), p["conv1"]["W"], row(p["conv1"]["b"]))

    ep1 = _SEG_E(xt1, nidx_g, eidx_s)
    e1 = _tc_edge(ep1, dp_e, None, p["conv1"]["w_att"], row(p["conv1"]["b_att"]))
    vp1 = _SEG_V(e1, eidx_g, nidx_s)
    xt2 = _tc_node(vp1, dp_v, xt1, p["conv2"]["W"], row(p["conv2"]["b"]))

    ep2 = _SEG_E(xt2, nidx_g, eidx_s)
    e2 = _tc_edge(ep2, dp_e, e1, p["conv2"]["w_att"], row(p["conv2"]["b_att"]))
    vp2 = _SEG_V(e2, eidx_g, nidx_s)
    xt3 = _tc_node(vp2, dp_v, xt2, p["conv3"]["W"], row(p["conv3"]["b"]))

    ep3 = _SEG_E(xt3, nidx_g, eidx_s)
    score, fatt = _tc_final(ep3, dp_e, e2, p["conv3"]["w_att"],
                            row(p["conv3"]["b_att"]), p)
    return score[:N_EDGES], fatt[:N_EDGES]


# speedup vs baseline: 1.5050x; 1.0076x over previous
import functools

import jax
import jax.numpy as jnp
from jax import lax
from jax.experimental import pallas as pl
from jax.experimental.pallas import tpu as pltpu
from jax.experimental.pallas import tpu_sc as plsc

N_NODES = 10000
N_EDGES = 2500
NNZ = 320000
D = 128

NC, NS = 2, 16
NW = NC * NS
PER_W = NNZ // NW
CH_E = 80
NCH_E = -(-PER_W // CH_E)
CH_V = 80
NCH_V = -(-PER_W // CH_V)
E_PAD = 2560
V_PAD = 10112
DEG_W = 16

_MESH = plsc.VectorSubcoreMesh(core_axis_name="c", subcore_axis_name="s")


def _zero_stripe(buf, ch, d, acc, base, stripe):
    zeros = jnp.zeros((16,), jnp.float32)

    def zrow(i, carry):
        for j in range(d // 16):
            buf[i, pl.ds(j * 16, 16)] = zeros
        return carry

    lax.fori_loop(0, ch, zrow, 0)
    off = 0
    while off < stripe:
        n = min(ch, stripe - off)
        pltpu.sync_copy(buf.at[pl.ds(0, n)], acc.at[pl.ds(base + off, n)])
        off += n


def _seg_loop(tbl, gv, sv, buf_a, buf_b, acc, sem_a, sem_b, nch):
    pltpu.async_copy(tbl.at[gv.at[0]], buf_a, sem_a)

    def chunk(i, carry):
        j0 = i * 2
        pltpu.async_copy(tbl.at[gv.at[j0 + 1]], buf_b, sem_b)
        pltpu.make_async_copy(tbl.at[gv.at[j0]], buf_a, sem_a).wait()
        pltpu.sync_copy(buf_a, acc.at[sv.at[j0]], add=True)

        @pl.when(j0 + 2 < nch)
        def _():
            pltpu.async_copy(tbl.at[gv.at[j0 + 2]], buf_a, sem_a)

        pltpu.make_async_copy(tbl.at[gv.at[j0 + 1]], buf_b, sem_b).wait()
        pltpu.sync_copy(buf_b, acc.at[sv.at[j0 + 1]], add=True)
        return carry

    lax.fori_loop(0, nch // 2, chunk, 0)
    if nch % 2:
        j = nch - 1
        pltpu.make_async_copy(tbl.at[gv.at[j]], buf_a, sem_a).wait()
        pltpu.sync_copy(buf_a, acc.at[sv.at[j]], add=True)


def _seg_loop_ring(tbl, gv, sv, bufs, sems_g, sems_s, acc, nch):
    nb = len(bufs)
    pltpu.async_copy(tbl.at[gv.at[0]], bufs[0], sems_g[0])
    nsteps = -(-nch // nb)

    def step(i, carry):
        for p in range(nb):
            t = i * nb + p
            pn = (p + 1) % nb

            @pl.when(t < nch)
            def _():
                @pl.when(jnp.logical_and(t + 1 < nch, t - (nb - 1) >= 0))
                def _():
                    pltpu.make_async_copy(
                        bufs[pn], acc.at[sv.at[t - (nb - 1)]],
                        sems_s[pn]).wait()

                @pl.when(t + 1 < nch)
                def _():
                    pltpu.async_copy(tbl.at[gv.at[t + 1]], bufs[pn],
                                     sems_g[pn])

                pltpu.make_async_copy(tbl.at[gv.at[t]], bufs[p],
                                      sems_g[p]).wait()
                pltpu.async_copy(bufs[p], acc.at[sv.at[t]], sems_s[p],
                                 add=True)
        return carry

    lax.fori_loop(0, nsteps, step, 0)
    for k in range(max(0, nch - nb), nch):
        pltpu.make_async_copy(bufs[k % nb], acc.at[sv.at[k]],
                              sems_s[k % nb]).wait()


def _make_seg_sum(n_pad, nch, ch, nb):
    stripe = n_pad // NS

    @functools.partial(
        pl.kernel,
        out_type=jax.ShapeDtypeStruct((NC, n_pad, D), jnp.float32),
        mesh=_MESH,
        scratch_types=(
            [pltpu.VMEM((nch, ch), jnp.int32),
             pltpu.VMEM((nch, ch), jnp.int32)]
            + [pltpu.VMEM((ch, D), jnp.float32)] * nb
            + [pltpu.VMEM_SHARED((n_pad, D), jnp.float32)]
            + [pltpu.SemaphoreType.DMA] * (2 * nb)
        ),
        compiler_params=pltpu.CompilerParams(use_tc_tiling_on_sc=False),
    )
    def seg_sum(table_hbm, gidx_hbm, sidx_hbm, out_hbm, gv, sv, *rest):
        bufs = rest[:nb]
        acc = rest[nb]
        sems_g = rest[nb + 1:nb + 1 + nb]
        sems_s = rest[nb + 1 + nb:]
        c = lax.axis_index("c")
        s = lax.axis_index("s")
        pltpu.sync_copy(gidx_hbm.at[c, s], gv)
        pltpu.sync_copy(sidx_hbm.at[c, s], sv)
        base = s * stripe
        _zero_stripe(bufs[0], ch, D, acc, base, stripe)
        plsc.subcore_barrier()
        if nb == 2:
            _seg_loop(table_hbm, gv, sv, bufs[0], bufs[1], acc,
                      sems_g[0], sems_g[1], nch)
        else:
            _seg_loop_ring(table_hbm, gv, sv, bufs, sems_g, sems_s, acc, nch)
        plsc.subcore_barrier()
        pltpu.sync_copy(acc.at[pl.ds(base, stripe)],
                        out_hbm.at[c, pl.ds(base, stripe)])

    return seg_sum


_SEG_E = _make_seg_sum(E_PAD, NCH_E, CH_E, 2)
_SEG_V = _make_seg_sum(V_PAD, NCH_V, CH_V, 2)


@functools.partial(
    pl.kernel,
    out_type=(jax.ShapeDtypeStruct((NC, E_PAD, D), jnp.float32),
              jax.ShapeDtypeStruct((NC, E_PAD, DEG_W), jnp.float32),
              jax.ShapeDtypeStruct((NC, V_PAD, DEG_W), jnp.float32)),
    mesh=_MESH,
    scratch_types=[
        pltpu.VMEM((NCH_E, CH_E), jnp.int32),
        pltpu.VMEM((NCH_E, CH_E), jnp.int32),
        pltpu.VMEM((CH_E, D), jnp.float32),
        pltpu.VMEM((CH_E, D), jnp.float32),
        pltpu.VMEM((CH_E, DEG_W), jnp.float32),
        pltpu.VMEM((CH_E, DEG_W), jnp.float32),
        pltpu.VMEM_SHARED((E_PAD, D), jnp.float32),
        pltpu.VMEM_SHARED((E_PAD, DEG_W), jnp.float32),
        pltpu.VMEM_SHARED((V_PAD, DEG_W), jnp.float32),
        pltpu.SemaphoreType.DMA,
        pltpu.SemaphoreType.DMA,
    ],
    compiler_params=pltpu.CompilerParams(use_tc_tiling_on_sc=False),
)
def _SEG_E_DEG(table_hbm, gidx_hbm, sidx_hbm, out_hbm, oute_hbm, outv_hbm,
               gv, sv, buf_a, buf_b, ones_b, zero_b, acc, acc_e, acc_v,
               sem_a, sem_b):
    c = lax.axis_index("c")
    s = lax.axis_index("s")
    pltpu.sync_copy(gidx_hbm.at[c, s], gv)
    pltpu.sync_copy(sidx_hbm.at[c, s], sv)
    stripe = E_PAD // NS
    base = s * stripe
    _zero_stripe(buf_a, CH_E, D, acc, base, stripe)

    ones = jnp.ones((16,), jnp.float32)
    zeros = jnp.zeros((16,), jnp.float32)

    def fill(i, carry):
        ones_b[i, pl.ds(0, DEG_W)] = ones
        zero_b[i, pl.ds(0, DEG_W)] = zeros
        return carry

    lax.fori_loop(0, CH_E, fill, 0)
    sv_ = V_PAD // NS
    for dbase, dstripe, dacc in ((base, stripe, acc_e),
                                 (s * sv_, sv_, acc_v)):
        off = 0
        while off < dstripe:
            n = min(CH_E, dstripe - off)
            pltpu.sync_copy(zero_b.at[pl.ds(0, n)],
                            dacc.at[pl.ds(dbase + off, n)])
            off += n
    plsc.subcore_barrier()

    pltpu.async_copy(table_hbm.at[gv.at[0]], buf_a, sem_a)

    def chunk(i, carry):
        for p, (bc, bn, sc_, sn) in enumerate(
                ((buf_a, buf_b, sem_a, sem_b), (buf_b, buf_a, sem_b, sem_a))):
            j = i * 2 + p

            @pl.when(j + 1 < NCH_E)
            def _():
                pltpu.async_copy(table_hbm.at[gv.at[j + 1]], bn, sn)

            pltpu.make_async_copy(table_hbm.at[gv.at[j]], bc, sc_).wait()
            pltpu.sync_copy(bc, acc.at[sv.at[j]], add=True)
            pltpu.sync_copy(ones_b, acc_e.at[sv.at[j]], add=True)
            pltpu.sync_copy(ones_b, acc_v.at[gv.at[j]], add=True)
        return carry

    lax.fori_loop(0, NCH_E // 2, chunk, 0)
    if NCH_E % 2:
        j = NCH_E - 1
        pltpu.make_async_copy(table_hbm.at[gv.at[j]], buf_a, sem_a).wait()
        pltpu.sync_copy(buf_a, acc.at[sv.at[j]], add=True)
        pltpu.sync_copy(ones_b, acc_e.at[sv.at[j]], add=True)
        pltpu.sync_copy(ones_b, acc_v.at[gv.at[j]], add=True)
    plsc.subcore_barrier()
    pltpu.sync_copy(acc.at[pl.ds(base, stripe)],
                    out_hbm.at[c, pl.ds(base, stripe)])
    pltpu.sync_copy(acc_e.at[pl.ds(base, stripe)],
                    oute_hbm.at[c, pl.ds(base, stripe)])
    pltpu.sync_copy(acc_v.at[pl.ds(s * sv_, sv_)],
                    outv_hbm.at[c, pl.ds(s * sv_, sv_)])


_NODE_BLK = 1000
_NODE_GRID = N_NODES // _NODE_BLK


def _tc_ft_body(x, wft, bft, lng, lnb, w1, b1, out):
    h = jnp.dot(x[...], wft[...], preferred_element_type=jnp.float32) + bft[...]
    h = jax.nn.gelu(h)
    m = jnp.mean(h, axis=-1, keepdims=True)
    var = jnp.mean((h - m) * (h - m), axis=-1, keepdims=True)
    h = (h - m) / jnp.sqrt(var + 1e-5) * lng[...] + lnb[...]
    out[...] = jnp.dot(h, w1[...], preferred_element_type=jnp.float32) + b1[...]


def _tc_ft(X, wft, bft, lng, lnb, w1, b1):
    full = lambda i: (0, 0)
    return pl.pallas_call(
        _tc_ft_body,
        grid=(_NODE_GRID,),
        in_specs=[
            pl.BlockSpec((_NODE_BLK, D), lambda i: (i, 0)),
            pl.BlockSpec((D, D), full),
            pl.BlockSpec((1, D), full),
            pl.BlockSpec((1, D), full),
            pl.BlockSpec((1, D), full),
            pl.BlockSpec((D, D), full),
            pl.BlockSpec((1, D), full),
        ],
        out_specs=pl.BlockSpec((_NODE_BLK, D), lambda i: (i, 0)),
        out_shape=jax.ShapeDtypeStruct((N_NODES, D), jnp.float32),
    )(X, wft, bft, lng, lnb, w1, b1)


def _tc_edge_body(has_prev, *refs):
    if has_prev:
        ep, dp, eprev, watt, batt, out = refs
    else:
        ep, dp, watt, batt, out = refs
    deg = jnp.clip(dp[0, :, 0:1] + dp[1, :, 0:1], 1.0, None)
    ef = (ep[0] + ep[1]) / deg
    if has_prev:
        ef = ef + eprev[...]
    a = jax.nn.sigmoid(
        jnp.dot(ef, watt[...], preferred_element_type=jnp.float32) + batt[...])
    out[...] = ef * a


def _tc_edge(ep, dp, eprev, watt, batt):
    args = [ep, dp] + ([eprev] if eprev is not None else []) + [watt, batt]
    return pl.pallas_call(
        functools.partial(_tc_edge_body, eprev is not None),
        out_shape=jax.ShapeDtypeStruct((E_PAD, D), jnp.float32),
    )(*args)


def _tc_node_body(vp, dvp, xt, w, b, out):
    deg = jnp.clip(dvp[0, :, 0:1] + dvp[1, :, 0:1], 1.0, None)
    v = (vp[0] + vp[1]) / deg + xt[...]
    v = jax.nn.gelu(v)
    out[...] = jnp.dot(v, w[...], preferred_element_type=jnp.float32) + b[...]


def _tc_node(vp, dvp, xt, w, b):
    full = lambda i: (0, 0)
    return pl.pallas_call(
        _tc_node_body,
        grid=(_NODE_GRID,),
        in_specs=[
            pl.BlockSpec((2, _NODE_BLK, D), lambda i: (0, i, 0)),
            pl.BlockSpec((2, _NODE_BLK, DEG_W), lambda i: (0, i, 0)),
            pl.BlockSpec((_NODE_BLK, D), lambda i: (i, 0)),
            pl.BlockSpec((D, D), full),
            pl.BlockSpec((1, D), full),
        ],
        out_specs=pl.BlockSpec((_NODE_BLK, D), lambda i: (i, 0)),
        out_shape=jax.ShapeDtypeStruct((N_NODES, D), jnp.float32),
    )(vp, dvp, xt, w, b)


def _tc_final_body(ep, dp, e2, watt, batt, wr, br, w1c, b1c, w2b, b2v,
                   wf1, bf1, wf2, bf2, bnm, bnv, bng, bnb, wo, bo,
                   score_out, att_out):
    deg = jnp.clip(dp[0, :, 0:1] + dp[1, :, 0:1], 1.0, None)
    ef = (ep[0] + ep[1]) / deg + e2[...]
    a3 = jax.nn.sigmoid(
        jnp.dot(ef, watt[...], preferred_element_type=jnp.float32) + batt[...])
    e3 = ef * a3
    refined = jax.nn.gelu(
        jnp.dot(e3, wr[...], preferred_element_type=jnp.float32) + br[...])
    t = jax.nn.gelu(
        jnp.dot(refined, w1c[...], preferred_element_type=jnp.float32) + b1c[...])
    combined = jnp.dot(t, w2b[...], preferred_element_type=jnp.float32) + b2v[...]
    aw = jax.nn.sigmoid(jnp.mean(combined, axis=1, keepdims=True))
    fatt = (aw + a3) * 0.5
    xw = refined * fatt
    t1 = jax.nn.gelu(
        jnp.dot(xw, wf1[...], preferred_element_type=jnp.float32) + bf1[...])
    xe = jax.nn.gelu(
        jnp.dot(t1, wf2[...], preferred_element_type=jnp.float32) + bf2[...])
    xs = xe + xw
    xs = (xs - bnm[...]) / jnp.sqrt(bnv[...] + 1e-5) * bng[...] + bnb[...]
    score_out[...] = jax.nn.sigmoid(
        jnp.dot(xs, wo[...], preferred_element_type=jnp.float32) + bo[...])
    att_out[...] = fatt


def _tc_final(ep, dp, e2, watt, batt, p):
    w1c = jnp.concatenate([hp["l1"]["W"] for hp in p["heads"]], axis=1)
    b1c = jnp.concatenate([hp["l1"]["b"] for hp in p["heads"]])[None, :]
    w2b = jax.scipy.linalg.block_diag(*[hp["l2"]["W"] for hp in p["heads"]])
    b2v = jnp.stack([hp["l2"]["b"][0] for hp in p["heads"]])[None, :]
    row = lambda a: a[None, :]
    return pl.pallas_call(
        _tc_final_body,
        out_shape=(jax.ShapeDtypeStruct((E_PAD, 1), jnp.float32),
                   jax.ShapeDtypeStruct((E_PAD, 1), jnp.float32)),
    )(ep, dp, e2, watt, batt,
      p["refine"]["W"], row(p["refine"]["b"]), w1c, b1c, w2b, b2v,
      p["fe1"]["W"], row(p["fe1"]["b"]), p["fe2"]["W"], row(p["fe2"]["b"]),
      row(p["bn_m"]), row(p["bn_v"]), row(p["bn_g"]), row(p["bn_b"]),
      p["out"]["W"], row(p["out"]["b"]))


def kernel(X, node_idx, edge_idx, params):
    p = params
    nidx = node_idx.astype(jnp.int32).reshape(NC, NS, NCH_E, CH_E)
    eidx = edge_idx.astype(jnp.int32).reshape(NC, NS, NCH_E, CH_E)
    row = lambda a: a[None, :]

    xt1 = _tc_ft(X, p["ft"]["W"], row(p["ft"]["b"]), row(p["ln_g"]),
                 row(p["ln_b"]), p["conv1"]["W"], row(p["conv1"]["b"]))

    ep1, dp_e, dp_v = _SEG_E_DEG(xt1, nidx, eidx)
    e1 = _tc_edge(ep1, dp_e, None, p["conv1"]["w_att"], row(p["conv1"]["b_att"]))
    vp1 = _SEG_V(e1, eidx, nidx)
    xt2 = _tc_node(vp1, dp_v, xt1, p["conv2"]["W"], row(p["conv2"]["b"]))

    ep2 = _SEG_E(xt2, nidx, eidx)
    e2 = _tc_edge(ep2, dp_e, e1, p["conv2"]["w_att"], row(p["conv2"]["b_att"]))
    vp2 = _SEG_V(e2, eidx, nidx)
    xt3 = _tc_node(vp2, dp_v, xt2, p["conv3"]["W"], row(p["conv3"]["b"]))

    ep3 = _SEG_E(xt3, nidx, eidx)
    score, fatt = _tc_final(ep3, dp_e, e2, p["conv3"]["w_att"],
                            row(p["conv3"]["b_att"]), p)
    return score[:N_EDGES], fatt[:N_EDGES]


# speedup vs baseline: 1.5430x; 1.0252x over previous
import functools

import jax
import jax.numpy as jnp
from jax import lax
from jax.experimental import pallas as pl
from jax.experimental.pallas import tpu as pltpu
from jax.experimental.pallas import tpu_sc as plsc

N_NODES = 10000
N_EDGES = 2500
NNZ = 320000
D = 128

NC, NS = 2, 16
NW = NC * NS
PER_W = NNZ // NW
CH_E = 80
NCH_E = -(-PER_W // CH_E)
CH_V = 80
NCH_V = -(-PER_W // CH_V)
E_PAD = 2560
V_PAD = 10112
DEG_W = 16

_MESH = plsc.VectorSubcoreMesh(core_axis_name="c", subcore_axis_name="s")


def _zero_stripe(buf, ch, d, acc, base, stripe):
    zeros = jnp.zeros((16,), jnp.float32)

    def zrow(i, carry):
        for j in range(d // 16):
            buf[i, pl.ds(j * 16, 16)] = zeros
        return carry

    lax.fori_loop(0, ch, zrow, 0)
    off = 0
    while off < stripe:
        n = min(ch, stripe - off)
        pltpu.sync_copy(buf.at[pl.ds(0, n)], acc.at[pl.ds(base + off, n)])
        off += n


def _seg_loop(tbl, gv, sv, buf_a, buf_b, acc, sem_a, sem_b, nch):
    pltpu.async_copy(tbl.at[gv.at[0]], buf_a, sem_a)

    def chunk(i, carry):
        j0 = i * 2
        pltpu.async_copy(tbl.at[gv.at[j0 + 1]], buf_b, sem_b)
        pltpu.make_async_copy(tbl.at[gv.at[j0]], buf_a, sem_a).wait()
        pltpu.sync_copy(buf_a, acc.at[sv.at[j0]], add=True)

        @pl.when(j0 + 2 < nch)
        def _():
            pltpu.async_copy(tbl.at[gv.at[j0 + 2]], buf_a, sem_a)

        pltpu.make_async_copy(tbl.at[gv.at[j0 + 1]], buf_b, sem_b).wait()
        pltpu.sync_copy(buf_b, acc.at[sv.at[j0 + 1]], add=True)
        return carry

    lax.fori_loop(0, nch // 2, chunk, 0)
    if nch % 2:
        j = nch - 1
        pltpu.make_async_copy(tbl.at[gv.at[j]], buf_a, sem_a).wait()
        pltpu.sync_copy(buf_a, acc.at[sv.at[j]], add=True)


def _seg_loop_ring(tbl, gv, sv, bufs, sems_g, sems_s, acc, nch):
    nb = len(bufs)
    pltpu.async_copy(tbl.at[gv.at[0]], bufs[0], sems_g[0])
    nsteps = -(-nch // nb)

    def step(i, carry):
        for p in range(nb):
            t = i * nb + p
            pn = (p + 1) % nb

            @pl.when(t < nch)
            def _():
                @pl.when(jnp.logical_and(t + 1 < nch, t - (nb - 1) >= 0))
                def _():
                    pltpu.make_async_copy(
                        bufs[pn], acc.at[sv.at[t - (nb - 1)]],
                        sems_s[pn]).wait()

                @pl.when(t + 1 < nch)
                def _():
                    pltpu.async_copy(tbl.at[gv.at[t + 1]], bufs[pn],
                                     sems_g[pn])

                pltpu.make_async_copy(tbl.at[gv.at[t]], bufs[p],
                                      sems_g[p]).wait()
                pltpu.async_copy(bufs[p], acc.at[sv.at[t]], sems_s[p],
                                 add=True)
        return carry

    lax.fori_loop(0, nsteps, step, 0)
    for k in range(max(0, nch - nb), nch):
        pltpu.make_async_copy(bufs[k % nb], acc.at[sv.at[k]],
                              sems_s[k % nb]).wait()


def _make_seg_sum(n_pad, nch, ch, nb):
    stripe = n_pad // NS

    @functools.partial(
        pl.kernel,
        out_type=jax.ShapeDtypeStruct((NC, n_pad, D), jnp.float32),
        mesh=_MESH,
        scratch_types=(
            [pltpu.VMEM((nch, ch), jnp.int32),
             pltpu.VMEM((nch, ch), jnp.int32)]
            + [pltpu.VMEM((ch, D), jnp.float32)] * nb
            + [pltpu.VMEM_SHARED((n_pad, D), jnp.float32)]
            + [pltpu.SemaphoreType.DMA] * (2 * nb)
        ),
        compiler_params=pltpu.CompilerParams(use_tc_tiling_on_sc=False),
    )
    def seg_sum(table_hbm, gidx_hbm, sidx_hbm, out_hbm, gv, sv, *rest):
        bufs = rest[:nb]
        acc = rest[nb]
        sems_g = rest[nb + 1:nb + 1 + nb]
        sems_s = rest[nb + 1 + nb:]
        c = lax.axis_index("c")
        s = lax.axis_index("s")
        pltpu.sync_copy(gidx_hbm.at[c, s], gv)
        pltpu.sync_copy(sidx_hbm.at[c, s], sv)
        base = s * stripe
        _zero_stripe(bufs[0], ch, D, acc, base, stripe)
        plsc.subcore_barrier()
        if nb == 2:
            _seg_loop(table_hbm, gv, sv, bufs[0], bufs[1], acc,
                      sems_g[0], sems_g[1], nch)
        else:
            _seg_loop_ring(table_hbm, gv, sv, bufs, sems_g, sems_s, acc, nch)
        plsc.subcore_barrier()
        pltpu.sync_copy(acc.at[pl.ds(base, stripe)],
                        out_hbm.at[c, pl.ds(base, stripe)])

    return seg_sum


_SEG_E = _make_seg_sum(E_PAD, NCH_E, CH_E, 2)
_SEG_V = _make_seg_sum(V_PAD, NCH_V, CH_V, 2)


@functools.partial(
    pl.kernel,
    out_type=(jax.ShapeDtypeStruct((NC, E_PAD, D), jnp.float32),
              jax.ShapeDtypeStruct((NC, E_PAD, DEG_W), jnp.float32),
              jax.ShapeDtypeStruct((NC, V_PAD, DEG_W), jnp.float32)),
    mesh=_MESH,
    scratch_types=[
        pltpu.VMEM((NCH_E, CH_E), jnp.int32),
        pltpu.VMEM((NCH_E, CH_E), jnp.int32),
        pltpu.VMEM((CH_E, D), jnp.float32),
        pltpu.VMEM((CH_E, D), jnp.float32),
        pltpu.VMEM((CH_E, DEG_W), jnp.float32),
        pltpu.VMEM((CH_E, DEG_W), jnp.float32),
        pltpu.VMEM_SHARED((E_PAD, D), jnp.float32),
        pltpu.VMEM_SHARED((E_PAD, DEG_W), jnp.float32),
        pltpu.VMEM_SHARED((V_PAD, DEG_W), jnp.float32),
        pltpu.SemaphoreType.DMA,
        pltpu.SemaphoreType.DMA,
        pltpu.SemaphoreType.DMA,
        pltpu.SemaphoreType.DMA,
    ],
    compiler_params=pltpu.CompilerParams(use_tc_tiling_on_sc=False),
)
def _SEG_E_DEG(table_hbm, gidx_hbm, sidx_hbm, out_hbm, oute_hbm, outv_hbm,
               gv, sv, buf_a, buf_b, ones_b, zero_b, acc, acc_e, acc_v,
               sem_a, sem_b, sem_de, sem_dv):
    c = lax.axis_index("c")
    s = lax.axis_index("s")
    pltpu.sync_copy(gidx_hbm.at[c, s], gv)
    pltpu.sync_copy(sidx_hbm.at[c, s], sv)
    stripe = E_PAD // NS
    base = s * stripe
    _zero_stripe(buf_a, CH_E, D, acc, base, stripe)

    ones = jnp.ones((16,), jnp.float32)
    zeros = jnp.zeros((16,), jnp.float32)

    def fill(i, carry):
        ones_b[i, pl.ds(0, DEG_W)] = ones
        zero_b[i, pl.ds(0, DEG_W)] = zeros
        return carry

    lax.fori_loop(0, CH_E, fill, 0)
    sv_ = V_PAD // NS
    for dbase, dstripe, dacc in ((base, stripe, acc_e),
                                 (s * sv_, sv_, acc_v)):
        off = 0
        while off < dstripe:
            n = min(CH_E, dstripe - off)
            pltpu.sync_copy(zero_b.at[pl.ds(0, n)],
                            dacc.at[pl.ds(dbase + off, n)])
            off += n
    plsc.subcore_barrier()

    pltpu.async_copy(table_hbm.at[gv.at[0]], buf_a, sem_a)
    pltpu.async_copy(ones_b, acc_e.at[sv.at[0]], sem_de, add=True)
    pltpu.async_copy(ones_b, acc_v.at[gv.at[0]], sem_dv, add=True)

    def chunk(i, carry):
        for p, (bc, bn, sc_, sn) in enumerate(
                ((buf_a, buf_b, sem_a, sem_b), (buf_b, buf_a, sem_b, sem_a))):
            j = i * 2 + p

            @pl.when(j + 1 < NCH_E)
            def _():
                pltpu.async_copy(table_hbm.at[gv.at[j + 1]], bn, sn)

            pltpu.make_async_copy(table_hbm.at[gv.at[j]], bc, sc_).wait()
            pltpu.sync_copy(bc, acc.at[sv.at[j]], add=True)
            pltpu.make_async_copy(ones_b, acc_e.at[sv.at[j]], sem_de).wait()
            pltpu.make_async_copy(ones_b, acc_v.at[gv.at[j]], sem_dv).wait()

            @pl.when(j + 1 < NCH_E)
            def _():
                pltpu.async_copy(ones_b, acc_e.at[sv.at[j + 1]], sem_de,
                                 add=True)
                pltpu.async_copy(ones_b, acc_v.at[gv.at[j + 1]], sem_dv,
                                 add=True)
        return carry

    lax.fori_loop(0, NCH_E // 2, chunk, 0)
    if NCH_E % 2:
        j = NCH_E - 1
        pltpu.make_async_copy(table_hbm.at[gv.at[j]], buf_a, sem_a).wait()
        pltpu.sync_copy(buf_a, acc.at[sv.at[j]], add=True)
        pltpu.make_async_copy(ones_b, acc_e.at[sv.at[j]], sem_de).wait()
        pltpu.make_async_copy(ones_b, acc_v.at[gv.at[j]], sem_dv).wait()
    plsc.subcore_barrier()
    pltpu.sync_copy(acc.at[pl.ds(base, stripe)],
                    out_hbm.at[c, pl.ds(base, stripe)])
    pltpu.sync_copy(acc_e.at[pl.ds(base, stripe)],
                    oute_hbm.at[c, pl.ds(base, stripe)])
    pltpu.sync_copy(acc_v.at[pl.ds(s * sv_, sv_)],
                    outv_hbm.at[c, pl.ds(s * sv_, sv_)])


_NODE_BLK = 1000
_NODE_GRID = N_NODES // _NODE_BLK


def _tc_ft_body(x, wft, bft, lng, lnb, w1, b1, out):
    h = jnp.dot(x[...], wft[...], preferred_element_type=jnp.float32) + bft[...]
    h = jax.nn.gelu(h)
    m = jnp.mean(h, axis=-1, keepdims=True)
    var = jnp.mean((h - m) * (h - m), axis=-1, keepdims=True)
    h = (h - m) / jnp.sqrt(var + 1e-5) * lng[...] + lnb[...]
    out[...] = jnp.dot(h, w1[...], preferred_element_type=jnp.float32) + b1[...]


def _tc_ft(X, wft, bft, lng, lnb, w1, b1):
    full = lambda i: (0, 0)
    return pl.pallas_call(
        _tc_ft_body,
        grid=(_NODE_GRID,),
        in_specs=[
            pl.BlockSpec((_NODE_BLK, D), lambda i: (i, 0)),
            pl.BlockSpec((D, D), full),
            pl.BlockSpec((1, D), full),
            pl.BlockSpec((1, D), full),
            pl.BlockSpec((1, D), full),
            pl.BlockSpec((D, D), full),
            pl.BlockSpec((1, D), full),
        ],
        out_specs=pl.BlockSpec((_NODE_BLK, D), lambda i: (i, 0)),
        out_shape=jax.ShapeDtypeStruct((N_NODES, D), jnp.float32),
    )(X, wft, bft, lng, lnb, w1, b1)


def _tc_edge_body(has_prev, *refs):
    if has_prev:
        ep, dp, eprev, watt, batt, out = refs
    else:
        ep, dp, watt, batt, out = refs
    deg = jnp.clip(dp[0, :, 0:1] + dp[1, :, 0:1], 1.0, None)
    ef = (ep[0] + ep[1]) / deg
    if has_prev:
        ef = ef + eprev[...]
    a = jax.nn.sigmoid(
        jnp.dot(ef, watt[...], preferred_element_type=jnp.float32) + batt[...])
    out[...] = ef * a


def _tc_edge(ep, dp, eprev, watt, batt):
    args = [ep, dp] + ([eprev] if eprev is not None else []) + [watt, batt]
    return pl.pallas_call(
        functools.partial(_tc_edge_body, eprev is not None),
        out_shape=jax.ShapeDtypeStruct((E_PAD, D), jnp.float32),
    )(*args)


def _tc_node_body(vp, dvp, xt, w, b, out):
    deg = jnp.clip(dvp[0, :, 0:1] + dvp[1, :, 0:1], 1.0, None)
    v = (vp[0] + vp[1]) / deg + xt[...]
    v = jax.nn.gelu(v)
    out[...] = jnp.dot(v, w[...], preferred_element_type=jnp.float32) + b[...]


def _tc_node(vp, dvp, xt, w, b):
    full = lambda i: (0, 0)
    return pl.pallas_call(
        _tc_node_body,
        grid=(_NODE_GRID,),
        in_specs=[
            pl.BlockSpec((2, _NODE_BLK, D), lambda i: (0, i, 0)),
            pl.BlockSpec((2, _NODE_BLK, DEG_W), lambda i: (0, i, 0)),
            pl.BlockSpec((_NODE_BLK, D), lambda i: (i, 0)),
            pl.BlockSpec((D, D), full),
            pl.BlockSpec((1, D), full),
        ],
        out_specs=pl.BlockSpec((_NODE_BLK, D), lambda i: (i, 0)),
        out_shape=jax.ShapeDtypeStruct((N_NODES, D), jnp.float32),
    )(vp, dvp, xt, w, b)


def _tc_final_body(ep, dp, e2, watt, batt, wr, br, w1c, b1c, w2b, b2v,
                   wf1, bf1, wf2, bf2, bnm, bnv, bng, bnb, wo, bo,
                   score_out, att_out):
    deg = jnp.clip(dp[0, :, 0:1] + dp[1, :, 0:1], 1.0, None)
    ef = (ep[0] + ep[1]) / deg + e2[...]
    a3 = jax.nn.sigmoid(
        jnp.dot(ef, watt[...], preferred_element_type=jnp.float32) + batt[...])
    e3 = ef * a3
    refined = jax.nn.gelu(
        jnp.dot(e3, wr[...], preferred_element_type=jnp.float32) + br[...])
    t = jax.nn.gelu(
        jnp.dot(refined, w1c[...], preferred_element_type=jnp.float32) + b1c[...])
    combined = jnp.dot(t, w2b[...], preferred_element_type=jnp.float32) + b2v[...]
    aw = jax.nn.sigmoid(jnp.mean(combined, axis=1, keepdims=True))
    fatt = (aw + a3) * 0.5
    xw = refined * fatt
    t1 = jax.nn.gelu(
        jnp.dot(xw, wf1[...], preferred_element_type=jnp.float32) + bf1[...])
    xe = jax.nn.gelu(
        jnp.dot(t1, wf2[...], preferred_element_type=jnp.float32) + bf2[...])
    xs = xe + xw
    xs = (xs - bnm[...]) / jnp.sqrt(bnv[...] + 1e-5) * bng[...] + bnb[...]
    score_out[...] = jax.nn.sigmoid(
        jnp.dot(xs, wo[...], preferred_element_type=jnp.float32) + bo[...])
    att_out[...] = fatt


def _tc_final(ep, dp, e2, watt, batt, p):
    w1c = jnp.concatenate([hp["l1"]["W"] for hp in p["heads"]], axis=1)
    b1c = jnp.concatenate([hp["l1"]["b"] for hp in p["heads"]])[None, :]
    w2b = jax.scipy.linalg.block_diag(*[hp["l2"]["W"] for hp in p["heads"]])
    b2v = jnp.stack([hp["l2"]["b"][0] for hp in p["heads"]])[None, :]
    row = lambda a: a[None, :]
    return pl.pallas_call(
        _tc_final_body,
        out_shape=(jax.ShapeDtypeStruct((E_PAD, 1), jnp.float32),
                   jax.ShapeDtypeStruct((E_PAD, 1), jnp.float32)),
    )(ep, dp, e2, watt, batt,
      p["refine"]["W"], row(p["refine"]["b"]), w1c, b1c, w2b, b2v,
      p["fe1"]["W"], row(p["fe1"]["b"]), p["fe2"]["W"], row(p["fe2"]["b"]),
      row(p["bn_m"]), row(p["bn_v"]), row(p["bn_g"]), row(p["bn_b"]),
      p["out"]["W"], row(p["out"]["b"]))


def kernel(X, node_idx, edge_idx, params):
    p = params
    nidx = node_idx.astype(jnp.int32).reshape(NC, NS, NCH_E, CH_E)
    eidx = edge_idx.astype(jnp.int32).reshape(NC, NS, NCH_E, CH_E)
    row = lambda a: a[None, :]

    xt1 = _tc_ft(X, p["ft"]["W"], row(p["ft"]["b"]), row(p["ln_g"]),
                 row(p["ln_b"]), p["conv1"]["W"], row(p["conv1"]["b"]))

    ep1, dp_e, dp_v = _SEG_E_DEG(xt1, nidx, eidx)
    e1 = _tc_edge(ep1, dp_e, None, p["conv1"]["w_att"], row(p["conv1"]["b_att"]))
    vp1 = _SEG_V(e1, eidx, nidx)
    xt2 = _tc_node(vp1, dp_v, xt1, p["conv2"]["W"], row(p["conv2"]["b"]))

    ep2 = _SEG_E(xt2, nidx, eidx)
    e2 = _tc_edge(ep2, dp_e, e1, p["conv2"]["w_att"], row(p["conv2"]["b_att"]))
    vp2 = _SEG_V(e2, eidx, nidx)
    xt3 = _tc_node(vp2, dp_v, xt2, p["conv3"]["W"], row(p["conv3"]["b"]))

    ep3 = _SEG_E(xt3, nidx, eidx)
    score, fatt = _tc_final(ep3, dp_e, e2, p["conv3"]["w_att"],
                            row(p["conv3"]["b_att"]), p)
    return score[:N_EDGES], fatt[:N_EDGES]


# speedup vs baseline: 1.5438x; 1.0005x over previous
import functools

import jax
import jax.numpy as jnp
from jax import lax
from jax.experimental import pallas as pl
from jax.experimental.pallas import tpu as pltpu
from jax.experimental.pallas import tpu_sc as plsc

N_NODES = 10000
N_EDGES = 2500
NNZ = 320000
D = 128

NC, NS = 2, 16
NW = NC * NS
PER_W = NNZ // NW
CH_E = 80
NCH_E = -(-PER_W // CH_E)
CH_V = 80
NCH_V = -(-PER_W // CH_V)
E_PAD = 2560
V_PAD = 10112
DEG_W = 16

_MESH = plsc.VectorSubcoreMesh(core_axis_name="c", subcore_axis_name="s")


def _zero_stripe(buf, ch, d, acc, base, stripe):
    zeros = jnp.zeros((16,), jnp.float32)

    def zrow(i, carry):
        for j in range(d // 16):
            buf[i, pl.ds(j * 16, 16)] = zeros
        return carry

    lax.fori_loop(0, ch, zrow, 0)
    off = 0
    while off < stripe:
        n = min(ch, stripe - off)
        pltpu.sync_copy(buf.at[pl.ds(0, n)], acc.at[pl.ds(base + off, n)])
        off += n


def _seg_loop(tbl, gv, sv, buf_a, buf_b, acc, sem_a, sem_b, nch):
    pltpu.async_copy(tbl.at[gv.at[0]], buf_a, sem_a)

    def chunk(i, carry):
        j0 = i * 2
        pltpu.async_copy(tbl.at[gv.at[j0 + 1]], buf_b, sem_b)
        pltpu.make_async_copy(tbl.at[gv.at[j0]], buf_a, sem_a).wait()
        pltpu.sync_copy(buf_a, acc.at[sv.at[j0]], add=True)

        @pl.when(j0 + 2 < nch)
        def _():
            pltpu.async_copy(tbl.at[gv.at[j0 + 2]], buf_a, sem_a)

        pltpu.make_async_copy(tbl.at[gv.at[j0 + 1]], buf_b, sem_b).wait()
        pltpu.sync_copy(buf_b, acc.at[sv.at[j0 + 1]], add=True)
        return carry

    lax.fori_loop(0, nch // 2, chunk, 0)
    if nch % 2:
        j = nch - 1
        pltpu.make_async_copy(tbl.at[gv.at[j]], buf_a, sem_a).wait()
        pltpu.sync_copy(buf_a, acc.at[sv.at[j]], add=True)


def _make_seg_sum(n_pad, nch, ch):
    stripe = n_pad // NS

    @functools.partial(
        pl.kernel,
        out_type=jax.ShapeDtypeStruct((NC, n_pad, D), jnp.float32),
        mesh=_MESH,
        scratch_types=[
            pltpu.VMEM((nch, ch), jnp.int32),
            pltpu.VMEM((nch, ch), jnp.int32),
            pltpu.VMEM((ch, D), jnp.float32),
            pltpu.VMEM((ch, D), jnp.float32),
            pltpu.VMEM_SHARED((n_pad, D), jnp.float32),
            pltpu.SemaphoreType.DMA,
            pltpu.SemaphoreType.DMA,
        ],
        compiler_params=pltpu.CompilerParams(use_tc_tiling_on_sc=False),
    )
    def seg_sum(table_hbm, gidx_hbm, sidx_hbm, out_hbm, gv, sv, buf_a, buf_b,
                acc, sem_a, sem_b):
        c = lax.axis_index("c")
        s = lax.axis_index("s")
        pltpu.sync_copy(gidx_hbm.at[c, s], gv)
        pltpu.sync_copy(sidx_hbm.at[c, s], sv)
        base = s * stripe
        _zero_stripe(buf_a, ch, D, acc, base, stripe)
        plsc.subcore_barrier()
        _seg_loop(table_hbm, gv, sv, buf_a, buf_b, acc, sem_a, sem_b, nch)
        plsc.subcore_barrier()
        pltpu.sync_copy(acc.at[pl.ds(base, stripe)],
                        out_hbm.at[c, pl.ds(base, stripe)])

    return seg_sum


_SEG_E = _make_seg_sum(E_PAD, NCH_E, CH_E)
_SEG_V = _make_seg_sum(V_PAD, NCH_V, CH_V)


@functools.partial(
    pl.kernel,
    out_type=(jax.ShapeDtypeStruct((NC, E_PAD, D), jnp.float32),
              jax.ShapeDtypeStruct((NC, E_PAD, DEG_W), jnp.float32),
              jax.ShapeDtypeStruct((NC, V_PAD, DEG_W), jnp.float32)),
    mesh=_MESH,
    scratch_types=[
        pltpu.VMEM((NCH_E, CH_E), jnp.int32),
        pltpu.VMEM((NCH_E, CH_E), jnp.int32),
        pltpu.VMEM((CH_E, D), jnp.float32),
        pltpu.VMEM((CH_E, D), jnp.float32),
        pltpu.VMEM((CH_E, DEG_W), jnp.float32),
        pltpu.VMEM((CH_E, DEG_W), jnp.float32),
        pltpu.VMEM_SHARED((E_PAD, D), jnp.float32),
        pltpu.VMEM_SHARED((E_PAD, DEG_W), jnp.float32),
        pltpu.VMEM_SHARED((V_PAD, DEG_W), jnp.float32),
        pltpu.SemaphoreType.DMA,
        pltpu.SemaphoreType.DMA,
        pltpu.SemaphoreType.DMA,
        pltpu.SemaphoreType.DMA,
    ],
    compiler_params=pltpu.CompilerParams(use_tc_tiling_on_sc=False),
)
def _SEG_E_DEG(table_hbm, gidx_hbm, sidx_hbm, out_hbm, oute_hbm, outv_hbm,
               gv, sv, buf_a, buf_b, ones_b, zero_b, acc, acc_e, acc_v,
               sem_a, sem_b, sem_de, sem_dv):
    c = lax.axis_index("c")
    s = lax.axis_index("s")
    pltpu.sync_copy(gidx_hbm.at[c, s], gv)
    pltpu.sync_copy(sidx_hbm.at[c, s], sv)
    stripe = E_PAD // NS
    base = s * stripe
    _zero_stripe(buf_a, CH_E, D, acc, base, stripe)

    ones = jnp.ones((16,), jnp.float32)
    zeros = jnp.zeros((16,), jnp.float32)

    def fill(i, carry):
        ones_b[i, pl.ds(0, DEG_W)] = ones
        zero_b[i, pl.ds(0, DEG_W)] = zeros
        return carry

    lax.fori_loop(0, CH_E, fill, 0)
    sv_ = V_PAD // NS
    for dbase, dstripe, dacc in ((base, stripe, acc_e),
                                 (s * sv_, sv_, acc_v)):
        off = 0
        while off < dstripe:
            n = min(CH_E, dstripe - off)
            pltpu.sync_copy(zero_b.at[pl.ds(0, n)],
                            dacc.at[pl.ds(dbase + off, n)])
            off += n
    plsc.subcore_barrier()

    pltpu.async_copy(table_hbm.at[gv.at[0]], buf_a, sem_a)
    pltpu.async_copy(ones_b, acc_e.at[sv.at[0]], sem_de, add=True)
    pltpu.async_copy(ones_b, acc_v.at[gv.at[0]], sem_dv, add=True)

    def chunk(i, carry):
        for p, (bc, bn, sc_, sn) in enumerate(
                ((buf_a, buf_b, sem_a, sem_b), (buf_b, buf_a, sem_b, sem_a))):
            j = i * 2 + p

            @pl.when(j + 1 < NCH_E)
            def _():
                pltpu.async_copy(table_hbm.at[gv.at[j + 1]], bn, sn)

            pltpu.make_async_copy(table_hbm.at[gv.at[j]], bc, sc_).wait()
            pltpu.sync_copy(bc, acc.at[sv.at[j]], add=True)
            pltpu.make_async_copy(ones_b, acc_e.at[sv.at[j]], sem_de).wait()
            pltpu.make_async_copy(ones_b, acc_v.at[gv.at[j]], sem_dv).wait()

            @pl.when(j + 1 < NCH_E)
            def _():
                pltpu.async_copy(ones_b, acc_e.at[sv.at[j + 1]], sem_de,
                                 add=True)
                pltpu.async_copy(ones_b, acc_v.at[gv.at[j + 1]], sem_dv,
                                 add=True)
        return carry

    lax.fori_loop(0, NCH_E // 2, chunk, 0)
    if NCH_E % 2:
        j = NCH_E - 1
        pltpu.make_async_copy(table_hbm.at[gv.at[j]], buf_a, sem_a).wait()
        pltpu.sync_copy(buf_a, acc.at[sv.at[j]], add=True)
        pltpu.make_async_copy(ones_b, acc_e.at[sv.at[j]], sem_de).wait()
        pltpu.make_async_copy(ones_b, acc_v.at[gv.at[j]], sem_dv).wait()
    plsc.subcore_barrier()
    pltpu.sync_copy(acc.at[pl.ds(base, stripe)],
                    out_hbm.at[c, pl.ds(base, stripe)])
    pltpu.sync_copy(acc_e.at[pl.ds(base, stripe)],
                    oute_hbm.at[c, pl.ds(base, stripe)])
    pltpu.sync_copy(acc_v.at[pl.ds(s * sv_, sv_)],
                    outv_hbm.at[c, pl.ds(s * sv_, sv_)])


_NODE_BLK = 1000
_NODE_GRID = N_NODES // _NODE_BLK


def _tc_ft_body(x, wft, bft, lng, lnb, w1, b1, out):
    h = jnp.dot(x[...], wft[...], preferred_element_type=jnp.float32) + bft[...]
    h = jax.nn.gelu(h)
    m = jnp.mean(h, axis=-1, keepdims=True)
    var = jnp.mean((h - m) * (h - m), axis=-1, keepdims=True)
    h = (h - m) / jnp.sqrt(var + 1e-5) * lng[...] + lnb[...]
    out[...] = jnp.dot(h, w1[...], preferred_element_type=jnp.float32) + b1[...]


def _tc_ft(X, wft, bft, lng, lnb, w1, b1):
    full = lambda i: (0, 0)
    return pl.pallas_call(
        _tc_ft_body,
        grid=(_NODE_GRID,),
        in_specs=[
            pl.BlockSpec((_NODE_BLK, D), lambda i: (i, 0)),
            pl.BlockSpec((D, D), full),
            pl.BlockSpec((1, D), full),
            pl.BlockSpec((1, D), full),
            pl.BlockSpec((1, D), full),
            pl.BlockSpec((D, D), full),
            pl.BlockSpec((1, D), full),
        ],
        out_specs=pl.BlockSpec((_NODE_BLK, D), lambda i: (i, 0)),
        out_shape=jax.ShapeDtypeStruct((N_NODES, D), jnp.float32),
    )(X, wft, bft, lng, lnb, w1, b1)


def _tc_edge_body(has_prev, *refs):
    if has_prev:
        ep, dp, eprev, watt, batt, out = refs
    else:
        ep, dp, watt, batt, out = refs
    deg = jnp.clip(dp[0, :, 0:1] + dp[1, :, 0:1], 1.0, None)
    ef = (ep[0] + ep[1]) / deg
    if has_prev:
        ef = ef + eprev[...]
    a = jax.nn.sigmoid(
        jnp.dot(ef, watt[...], preferred_element_type=jnp.float32) + batt[...])
    out[...] = ef * a


def _tc_edge(ep, dp, eprev, watt, batt):
    args = [ep, dp] + ([eprev] if eprev is not None else []) + [watt, batt]
    return pl.pallas_call(
        functools.partial(_tc_edge_body, eprev is not None),
        out_shape=jax.ShapeDtypeStruct((E_PAD, D), jnp.float32),
    )(*args)


def _tc_node_body(vp, dvp, xt, w, b, out):
    deg = jnp.clip(dvp[0, :, 0:1] + dvp[1, :, 0:1], 1.0, None)
    v = (vp[0] + vp[1]) / deg + xt[...]
    v = jax.nn.gelu(v)
    out[...] = jnp.dot(v, w[...], preferred_element_type=jnp.float32) + b[...]


def _tc_node(vp, dvp, xt, w, b):
    full = lambda i: (0, 0)
    return pl.pallas_call(
        _tc_node_body,
        grid=(_NODE_GRID,),
        in_specs=[
            pl.BlockSpec((2, _NODE_BLK, D), lambda i: (0, i, 0)),
            pl.BlockSpec((2, _NODE_BLK, DEG_W), lambda i: (0, i, 0)),
            pl.BlockSpec((_NODE_BLK, D), lambda i: (i, 0)),
            pl.BlockSpec((D, D), full),
            pl.BlockSpec((1, D), full),
        ],
        out_specs=pl.BlockSpec((_NODE_BLK, D), lambda i: (i, 0)),
        out_shape=jax.ShapeDtypeStruct((N_NODES, D), jnp.float32),
    )(vp, dvp, xt, w, b)


def _tc_final_body(ep, dp, e2, watt, batt, wr, br, w1c, b1c, w2b, b2v,
                   wf1, bf1, wf2, bf2, bnm, bnv, bng, bnb, wo, bo,
                   score_out, att_out):
    deg = jnp.clip(dp[0, :, 0:1] + dp[1, :, 0:1], 1.0, None)
    ef = (ep[0] + ep[1]) / deg + e2[...]
    a3 = jax.nn.sigmoid(
        jnp.dot(ef, watt[...], preferred_element_type=jnp.float32) + batt[...])
    e3 = ef * a3
    refined = jax.nn.gelu(
        jnp.dot(e3, wr[...], preferred_element_type=jnp.float32) + br[...])
    t = jax.nn.gelu(
        jnp.dot(refined, w1c[...], preferred_element_type=jnp.float32) + b1c[...])
    combined = jnp.dot(t, w2b[...], preferred_element_type=jnp.float32) + b2v[...]
    aw = jax.nn.sigmoid(jnp.mean(combined, axis=1, keepdims=True))
    fatt = (aw + a3) * 0.5
    xw = refined * fatt
    t1 = jax.nn.gelu(
        jnp.dot(xw, wf1[...], preferred_element_type=jnp.float32) + bf1[...])
    xe = jax.nn.gelu(
        jnp.dot(t1, wf2[...], preferred_element_type=jnp.float32) + bf2[...])
    xs = xe + xw
    xs = (xs - bnm[...]) / jnp.sqrt(bnv[...] + 1e-5) * bng[...] + bnb[...]
    score_out[...] = jax.nn.sigmoid(
        jnp.dot(xs, wo[...], preferred_element_type=jnp.float32) + bo[...])
    att_out[...] = fatt


def _tc_final(ep, dp, e2, watt, batt, p):
    w1c = jnp.concatenate([hp["l1"]["W"] for hp in p["heads"]], axis=1)
    b1c = jnp.concatenate([hp["l1"]["b"] for hp in p["heads"]])[None, :]
    w2b = jax.scipy.linalg.block_diag(*[hp["l2"]["W"] for hp in p["heads"]])
    b2v = jnp.stack([hp["l2"]["b"][0] for hp in p["heads"]])[None, :]
    row = lambda a: a[None, :]
    return pl.pallas_call(
        _tc_final_body,
        out_shape=(jax.ShapeDtypeStruct((E_PAD, 1), jnp.float32),
                   jax.ShapeDtypeStruct((E_PAD, 1), jnp.float32)),
    )(ep, dp, e2, watt, batt,
      p["refine"]["W"], row(p["refine"]["b"]), w1c, b1c, w2b, b2v,
      p["fe1"]["W"], row(p["fe1"]["b"]), p["fe2"]["W"], row(p["fe2"]["b"]),
      row(p["bn_m"]), row(p["bn_v"]), row(p["bn_g"]), row(p["bn_b"]),
      p["out"]["W"], row(p["out"]["b"]))


def kernel(X, node_idx, edge_idx, params):
    p = params
    nidx = node_idx.astype(jnp.int32).reshape(NC, NS, NCH_E, CH_E)
    eidx = edge_idx.astype(jnp.int32).reshape(NC, NS, NCH_E, CH_E)
    row = lambda a: a[None, :]

    xt1 = _tc_ft(X, p["ft"]["W"], row(p["ft"]["b"]), row(p["ln_g"]),
                 row(p["ln_b"]), p["conv1"]["W"], row(p["conv1"]["b"]))

    ep1, dp_e, dp_v = _SEG_E_DEG(xt1, nidx, eidx)
    e1 = _tc_edge(ep1, dp_e, None, p["conv1"]["w_att"], row(p["conv1"]["b_att"]))
    vp1 = _SEG_V(e1, eidx, nidx)
    xt2 = _tc_node(vp1, dp_v, xt1, p["conv2"]["W"], row(p["conv2"]["b"]))

    ep2 = _SEG_E(xt2, nidx, eidx)
    e2 = _tc_edge(ep2, dp_e, e1, p["conv2"]["w_att"], row(p["conv2"]["b_att"]))
    vp2 = _SEG_V(e2, eidx, nidx)
    xt3 = _tc_node(vp2, dp_v, xt2, p["conv3"]["W"], row(p["conv3"]["b"]))

    ep3 = _SEG_E(xt3, nidx, eidx)
    score, fatt = _tc_final(ep3, dp_e, e2, p["conv3"]["w_att"],
                            row(p["conv3"]["b_att"]), p)
    return score[:N_EDGES], fatt[:N_EDGES]
